# baseline (speedup 1.0000x reference)
import functools

import jax
import jax.numpy as jnp
from jax import lax
from jax.experimental import pallas as pl
from jax.experimental.pallas import tpu as pltpu

B, D, E, K = 4096, 1024, 8, 2
BM = 512
G = B // BM
H = G // 2
S = E + 1


def _moe_block(x_hbm, w_hbm, b_ref, gw_ref, gb_ref, out_hbm,
               acc_ref, xbf_ref, wmat_ref, xtmp_ref, wtmp_ref, wbf_ref,
               sx, sw, so):
    t = pl.program_id(0)

    def gate_block(g, slot):
        pltpu.make_async_copy(
            x_hbm.at[pl.ds(g * BM, BM)], xtmp_ref.at[slot], sx.at[slot]).wait()
        x32 = xtmp_ref[slot]
        logits = lax.dot_general(
            x32, gw_ref[...], (((1,), (1,)), ((), ())),
            preferred_element_type=jnp.float32) + gb_ref[...]
        cols = lax.broadcasted_iota(jnp.int32, logits.shape, 1)
        idx1 = jnp.argmax(logits, axis=1, keepdims=True)
        v1 = jnp.max(logits, axis=1, keepdims=True)
        l2 = jnp.where(cols == idx1, -jnp.inf, logits)
        idx2 = jnp.argmax(l2, axis=1, keepdims=True)
        v2 = jnp.max(l2, axis=1, keepdims=True)
        w1 = 1.0 / (1.0 + jnp.exp(v2 - v1))
        w_mat = jnp.where(cols == idx1, w1,
                          jnp.where(cols == idx2, 1.0 - w1, 0.0))
        wmat_ref[g * BM:(g + 1) * BM, :] = w_mat
        xbf_ref[g * BM:(g + 1) * BM, :] = x32.astype(jnp.bfloat16)
        acc_ref[g * BM:(g + 1) * BM, :] = lax.dot_general(
            w_mat, b_ref[...], (((1,), (0,)), ((), ())),
            preferred_element_type=jnp.float32)

    def expert_dot(g, e, fire):
        xbf = xbf_ref[g * BM:(g + 1) * BM, :]
        y = lax.dot_general(
            xbf, wbf_ref[...], (((1,), (1,)), ((), ())),
            preferred_element_type=jnp.float32)
        wm = wmat_ref[g * BM:(g + 1) * BM, :]
        ecols = lax.broadcasted_iota(jnp.int32, wm.shape, 1)
        w_col = jnp.sum(jnp.where(ecols == e, wm, 0.0),
                        axis=1, keepdims=True)
        acc_ref[g * BM:(g + 1) * BM, :] += w_col * y

        @pl.when(fire)
        def _fire():
            pltpu.make_async_copy(
                acc_ref.at[pl.ds(g * BM, BM)],
                out_hbm.at[pl.ds(g * BM, BM)], so).start()

    @pl.when(t == 0)
    def _front_gate():
        pltpu.make_async_copy(w_hbm.at[0], wtmp_ref.at[0], sw.at[0]).start()
        for g0 in range(H):
            pltpu.make_async_copy(
                x_hbm.at[pl.ds(g0 * BM, BM)],
                xtmp_ref.at[g0], sx.at[g0]).start()
        for g in range(H):
            gate_block(g, g)
            pltpu.make_async_copy(
                x_hbm.at[pl.ds((g + H) * BM, BM)],
                xtmp_ref.at[g], sx.at[g]).start()

    @pl.when(t == 1)
    def _back_gate():
        for g in range(H, G):
            gate_block(g, g - H)

    @pl.when(jnp.logical_and(t >= 1, t <= E))
    def _back_dots():
        for g in range(H, G):
            expert_dot(g, t - 1, t == E)

    @pl.when(t < E)
    def _w_arrive():
        ts = t % 2
        pltpu.make_async_copy(w_hbm.at[t], wtmp_ref.at[ts], sw.at[ts]).wait()
        wbf_ref[...] = wtmp_ref[ts].astype(jnp.bfloat16)

        @pl.when(t + 1 < E)
        def _w_prefetch():
            ns = (t + 1) % 2
            pltpu.make_async_copy(
                w_hbm.at[t + 1], wtmp_ref.at[ns], sw.at[ns]).start()

    @pl.when(t < E)
    def _front_dots():
        for g in range(H):
            expert_dot(g, t, t == E - 1)

    @pl.when(t == E)
    def _drain():
        for g in range(G):
            pltpu.make_async_copy(
                acc_ref.at[pl.ds(g * BM, BM)],
                out_hbm.at[pl.ds(g * BM, BM)], so).wait()


@functools.partial(jax.jit)
def _moe(x, W, b, gate_W, gate_b):
    return pl.pallas_call(
        _moe_block,
        grid=(S,),
        in_specs=[
            pl.BlockSpec(memory_space=pl.ANY),
            pl.BlockSpec(memory_space=pl.ANY),
            pl.BlockSpec((E, D), lambda t: (0, 0)),
            pl.BlockSpec((E, D), lambda t: (0, 0)),
            pl.BlockSpec((1, E), lambda t: (0, 0)),
        ],
        out_specs=pl.BlockSpec(memory_space=pl.ANY),
        out_shape=jax.ShapeDtypeStruct((B, D), jnp.float32),
        scratch_shapes=[
            pltpu.VMEM((B, D), jnp.float32),
            pltpu.VMEM((B, D), jnp.bfloat16),
            pltpu.VMEM((B, E), jnp.float32),
            pltpu.VMEM((H, BM, D), jnp.float32),
            pltpu.VMEM((2, D, D), jnp.float32),
            pltpu.VMEM((D, D), jnp.bfloat16),
            pltpu.SemaphoreType.DMA((H,)),
            pltpu.SemaphoreType.DMA((2,)),
            pltpu.SemaphoreType.DMA,
        ],
        compiler_params=pltpu.CompilerParams(
            dimension_semantics=("arbitrary",),
        ),
    )(x, W, b, gate_W, gate_b.reshape(1, E))


def kernel(x, W, b, gate_W, gate_b):
    return _moe(x, W, b, gate_W, gate_b)


# speedup vs baseline: 1.0101x; 1.0101x over previous
import functools

import jax
import jax.numpy as jnp
from jax import lax
from jax.experimental import pallas as pl
from jax.experimental.pallas import tpu as pltpu

B, D, E, K = 4096, 1024, 8, 2
BM = 512
G = B // BM
H = G // 2


def _moe_block(x_hbm, w_hbm, b_ref, gw_ref, gb_ref, out_hbm,
               acc_ref, xbf_ref, wmat_ref, xtmp_ref, wtmp_ref, wbf_ref,
               sx, sw, so):
    k = pl.program_id(0)

    def gate_block(g, slot):
        pltpu.make_async_copy(
            x_hbm.at[pl.ds(g * BM, BM)], xtmp_ref.at[slot], sx.at[slot]).wait()
        x32 = xtmp_ref[slot]
        logits = lax.dot_general(
            x32, gw_ref[...], (((1,), (1,)), ((), ())),
            preferred_element_type=jnp.float32) + gb_ref[...]
        cols = lax.broadcasted_iota(jnp.int32, logits.shape, 1)
        idx1 = jnp.argmax(logits, axis=1, keepdims=True)
        v1 = jnp.max(logits, axis=1, keepdims=True)
        l2 = jnp.where(cols == idx1, -jnp.inf, logits)
        idx2 = jnp.argmax(l2, axis=1, keepdims=True)
        v2 = jnp.max(l2, axis=1, keepdims=True)
        w1 = 1.0 / (1.0 + jnp.exp(v2 - v1))
        w_mat = jnp.where(cols == idx1, w1,
                          jnp.where(cols == idx2, 1.0 - w1, 0.0))
        wmat_ref[g * BM:(g + 1) * BM, :] = w_mat
        xbf_ref[g * BM:(g + 1) * BM, :] = x32.astype(jnp.bfloat16)
        acc_ref[g * BM:(g + 1) * BM, :] = lax.dot_general(
            w_mat, b_ref[...], (((1,), (0,)), ((), ())),
            preferred_element_type=jnp.float32)

    def expert_dot(g, e, buf, fire):
        xbf = xbf_ref[g * BM:(g + 1) * BM, :]
        y = lax.dot_general(
            xbf, wbf_ref[buf], (((1,), (1,)), ((), ())),
            preferred_element_type=jnp.float32)
        wm = wmat_ref[g * BM:(g + 1) * BM, :]
        ecols = lax.broadcasted_iota(jnp.int32, wm.shape, 1)
        w_col = jnp.sum(jnp.where(ecols == e, wm, 0.0),
                        axis=1, keepdims=True)
        acc_ref[g * BM:(g + 1) * BM, :] += w_col * y

        @pl.when(fire)
        def _fire():
            pltpu.make_async_copy(
                acc_ref.at[pl.ds(g * BM, BM)],
                out_hbm.at[pl.ds(g * BM, BM)], so).start()

    @pl.when(k == 0)
    def _even_gate():
        pltpu.make_async_copy(w_hbm.at[0], wtmp_ref.at[0], sw.at[0]).start()
        for g0 in range(H):
            pltpu.make_async_copy(
                x_hbm.at[pl.ds(g0 * BM, BM)],
                xtmp_ref.at[g0], sx.at[g0]).start()
        for g in range(H):
            gate_block(g, g)
            pltpu.make_async_copy(
                x_hbm.at[pl.ds((g + H) * BM, BM)],
                xtmp_ref.at[g], sx.at[g]).start()

    @pl.when(k <= 3)
    def _even_w():
        pltpu.make_async_copy(
            w_hbm.at[2 * k], wtmp_ref.at[0], sw.at[0]).wait()
        wbf_ref[0] = wtmp_ref[0].astype(jnp.bfloat16)
        pltpu.make_async_copy(
            w_hbm.at[2 * k + 1], wtmp_ref.at[1], sw.at[1]).start()

    @pl.when(jnp.logical_and(k >= 1, k <= 4))
    def _even_back():
        for g in range(H, G):
            expert_dot(g, 2 * k - 1, 1, k == 4)

    @pl.when(k <= 3)
    def _even_front():
        for g in range(H):
            expert_dot(g, 2 * k, 0, False)

    @pl.when(k == 0)
    def _odd_gate():
        for g in range(H, G):
            gate_block(g, g - H)

    @pl.when(k <= 3)
    def _odd_w():
        pltpu.make_async_copy(
            w_hbm.at[2 * k + 1], wtmp_ref.at[1], sw.at[1]).wait()
        wbf_ref[1] = wtmp_ref[1].astype(jnp.bfloat16)

        @pl.when(k <= 2)
        def _odd_prefetch():
            pltpu.make_async_copy(
                w_hbm.at[2 * k + 2], wtmp_ref.at[0], sw.at[0]).start()

    @pl.when(k <= 3)
    def _odd_back():
        for g in range(H, G):
            expert_dot(g, 2 * k, 0, False)

    @pl.when(k <= 3)
    def _odd_front():
        for g in range(H):
            expert_dot(g, 2 * k + 1, 1, k == 3)

    @pl.when(k == 4)
    def _drain():
        for g in range(G):
            pltpu.make_async_copy(
                acc_ref.at[pl.ds(g * BM, BM)],
                out_hbm.at[pl.ds(g * BM, BM)], so).wait()


@functools.partial(jax.jit)
def _moe(x, W, b, gate_W, gate_b):
    return pl.pallas_call(
        _moe_block,
        grid=(E // 2 + 1,),
        in_specs=[
            pl.BlockSpec(memory_space=pl.ANY),
            pl.BlockSpec(memory_space=pl.ANY),
            pl.BlockSpec((E, D), lambda k: (0, 0)),
            pl.BlockSpec((E, D), lambda k: (0, 0)),
            pl.BlockSpec((1, E), lambda k: (0, 0)),
        ],
        out_specs=pl.BlockSpec(memory_space=pl.ANY),
        out_shape=jax.ShapeDtypeStruct((B, D), jnp.float32),
        scratch_shapes=[
            pltpu.VMEM((B, D), jnp.float32),
            pltpu.VMEM((B, D), jnp.bfloat16),
            pltpu.VMEM((B, E), jnp.float32),
            pltpu.VMEM((H, BM, D), jnp.float32),
            pltpu.VMEM((2, D, D), jnp.float32),
            pltpu.VMEM((2, D, D), jnp.bfloat16),
            pltpu.SemaphoreType.DMA((H,)),
            pltpu.SemaphoreType.DMA((2,)),
            pltpu.SemaphoreType.DMA,
        ],
        compiler_params=pltpu.CompilerParams(
            dimension_semantics=("arbitrary",),
        ),
    )(x, W, b, gate_W, gate_b.reshape(1, E))


def kernel(x, W, b, gate_W, gate_b):
    return _moe(x, W, b, gate_W, gate_b)


# speedup vs baseline: 1.0315x; 1.0212x over previous
import functools

import jax
import jax.numpy as jnp
from jax import lax
from jax.experimental import pallas as pl
from jax.experimental.pallas import tpu as pltpu

B, D, E, K = 4096, 1024, 8, 2
BM = 512
G = B // BM


def _moe_block(x_hbm, w_hbm, b_ref, gw_ref, gb_ref, out_hbm,
               acc_ref, xbf_ref, wmat_ref, xtmp_ref, wtmp_ref, wbf_ref,
               sx, sw, so):
    e = pl.program_id(0)
    es = e % 2

    @pl.when(e == 0)
    def _w_prologue():
        pltpu.make_async_copy(w_hbm.at[0], wtmp_ref.at[0], sw.at[0]).start()

    pltpu.make_async_copy(w_hbm.at[e], wtmp_ref.at[es], sw.at[es]).wait()
    wbf_ref[...] = wtmp_ref[es].astype(jnp.bfloat16)

    @pl.when(e + 1 < E)
    def _w_prefetch():
        ns = (e + 1) % 2
        pltpu.make_async_copy(
            w_hbm.at[e + 1], wtmp_ref.at[ns], sw.at[ns]).start()

    @pl.when(e == 0)
    def _first_pass():
        for g0 in range(3):
            pltpu.make_async_copy(
                x_hbm.at[pl.ds(g0 * BM, BM)],
                xtmp_ref.at[g0], sx.at[g0]).start()
        for g in range(G):
            if g + 3 < G:
                pltpu.make_async_copy(
                    x_hbm.at[pl.ds((g + 3) * BM, BM)],
                    xtmp_ref.at[(g + 3) % 4], sx.at[(g + 3) % 4]).start()
            pltpu.make_async_copy(
                x_hbm.at[pl.ds(g * BM, BM)],
                xtmp_ref.at[g % 4], sx.at[g % 4]).wait()
            x32 = xtmp_ref[g % 4]

            logits = lax.dot_general(
                x32, gw_ref[...], (((1,), (1,)), ((), ())),
                preferred_element_type=jnp.float32) + gb_ref[...]
            cols = lax.broadcasted_iota(jnp.int32, logits.shape, 1)
            idx1 = jnp.argmax(logits, axis=1, keepdims=True)
            v1 = jnp.max(logits, axis=1, keepdims=True)
            l2 = jnp.where(cols == idx1, -jnp.inf, logits)
            idx2 = jnp.argmax(l2, axis=1, keepdims=True)
            v2 = jnp.max(l2, axis=1, keepdims=True)
            w1 = 1.0 / (1.0 + jnp.exp(v2 - v1))
            w_mat = jnp.where(cols == idx1, w1,
                              jnp.where(cols == idx2, 1.0 - w1, 0.0))
            wmat_ref[g * BM:(g + 1) * BM, :] = w_mat

            xbf = x32.astype(jnp.bfloat16)
            xbf_ref[g * BM:(g + 1) * BM, :] = xbf

            acc = lax.dot_general(
                w_mat, b_ref[...], (((1,), (0,)), ((), ())),
                preferred_element_type=jnp.float32)
            y = lax.dot_general(
                xbf, wbf_ref[...], (((1,), (1,)), ((), ())),
                preferred_element_type=jnp.float32)
            acc_ref[g * BM:(g + 1) * BM, :] = acc + w_mat[:, 0:1] * y

    @pl.when(e != 0)
    def _accumulate():
        for g in range(G):
            xbf = xbf_ref[g * BM:(g + 1) * BM, :]
            y = lax.dot_general(
                xbf, wbf_ref[...], (((1,), (1,)), ((), ())),
                preferred_element_type=jnp.float32)
            wm = wmat_ref[g * BM:(g + 1) * BM, :]
            ecols = lax.broadcasted_iota(jnp.int32, wm.shape, 1)
            w_col = jnp.sum(jnp.where(ecols == e, wm, 0.0),
                            axis=1, keepdims=True)
            acc_ref[g * BM:(g + 1) * BM, :] += w_col * y

            @pl.when(e == E - 1)
            def _fire(g=g):
                pltpu.make_async_copy(
                    acc_ref.at[pl.ds(g * BM, BM)],
                    out_hbm.at[pl.ds(g * BM, BM)], so).start()

    @pl.when(e == E - 1)
    def _writeback():
        for g in range(G):
            pltpu.make_async_copy(
                acc_ref.at[pl.ds(g * BM, BM)],
                out_hbm.at[pl.ds(g * BM, BM)], so).wait()


@functools.partial(jax.jit)
def _moe(x, W, b, gate_W, gate_b):
    return pl.pallas_call(
        _moe_block,
        grid=(E,),
        in_specs=[
            pl.BlockSpec(memory_space=pl.ANY),
            pl.BlockSpec(memory_space=pl.ANY),
            pl.BlockSpec((E, D), lambda e: (0, 0)),
            pl.BlockSpec((E, D), lambda e: (0, 0)),
            pl.BlockSpec((1, E), lambda e: (0, 0)),
        ],
        out_specs=pl.BlockSpec(memory_space=pl.ANY),
        out_shape=jax.ShapeDtypeStruct((B, D), jnp.float32),
        scratch_shapes=[
            pltpu.VMEM((B, D), jnp.float32),
            pltpu.VMEM((B, D), jnp.bfloat16),
            pltpu.VMEM((B, E), jnp.float32),
            pltpu.VMEM((4, BM, D), jnp.float32),
            pltpu.VMEM((2, D, D), jnp.float32),
            pltpu.VMEM((D, D), jnp.bfloat16),
            pltpu.SemaphoreType.DMA((4,)),
            pltpu.SemaphoreType.DMA((2,)),
            pltpu.SemaphoreType.DMA,
        ],
        compiler_params=pltpu.CompilerParams(
            dimension_semantics=("arbitrary",),
        ),
    )(x, W, b, gate_W, gate_b.reshape(1, E))


def kernel(x, W, b, gate_W, gate_b):
    return _moe(x, W, b, gate_W, gate_b)
